# baseline (speedup 1.0000x reference)
import functools

import jax
import jax.numpy as jnp
from jax import lax
from jax.experimental import pallas as pl
from jax.experimental.pallas import tpu as pltpu
from jax.experimental.pallas import tpu_sc as plsc

NN = 10000
NP = 10240
NE = 320000
EP = 327680
CH = 128
NSUB = 16
ROWS = NP // NSUB

_f32 = jnp.float32


def _make_prop(feat_split: bool, D: int, native_tiling: bool = False):
    mesh = plsc.VectorSubcoreMesh(core_axis_name="c", subcore_axis_name="s")
    out_type = [jax.ShapeDtypeStruct((NP, D), _f32)] * 2
    if feat_split:
        nch = EP // NSUB // CH
    else:
        nch = EP // (2 * NSUB) // CH
    BCH = 40
    nblk = nch // BCH
    scratch = [
        pltpu.VMEM_SHARED((NP, D), _f32),
        pltpu.VMEM((BCH, CH), jnp.int32),
        pltpu.VMEM((BCH, CH), jnp.int32),
        pltpu.VMEM((CH, D), _f32),
        pltpu.VMEM((CH, D), _f32),
        pltpu.SemaphoreType.DMA,
        pltpu.SemaphoreType.DMA,
        pltpu.SemaphoreType.DMA,
        pltpu.SemaphoreType.DMA,
    ]

    cp = (pltpu.CompilerParams(use_tc_tiling_on_sc=False)
          if native_tiling else None)

    @functools.partial(pl.kernel, out_type=out_type, mesh=mesh,
                       scratch_types=scratch, compiler_params=cp)
    def prop(h0, h1, src, dst, out0, out1,
             acc, sidx, didx, m0, m1, g0, g1, s0, s1):
        c = lax.axis_index("c")
        s = lax.axis_index("s")
        row0 = s * ROWS
        zv = jnp.zeros((16,), _f32)

        @pl.loop(0, CH)
        def _(r):
            @pl.loop(0, D, step=16)
            def _(k):
                m0[r, pl.ds(k, 16)] = zv

        @pl.loop(0, ROWS, step=CH)
        def _(r):
            pltpu.sync_copy(m0, acc.at[pl.ds(row0 + r, CH)])

        plsc.subcore_barrier()

        def edge_pass(h, chunk0):
            bufs = ((m0, g0, s0), (m1, g1, s1))

            def phase(j, cur, oth):
                mc, gc, sc = cur
                mo, go, so = oth
                pltpu.make_async_copy(h.at[sidx.at[j]], mc, gc).wait()

                @pl.when(j + 1 < BCH)
                def _():
                    @pl.when(j >= 1)
                    def _():
                        pltpu.make_async_copy(mo, acc.at[didx.at[0]],
                                              so).wait()
                    pltpu.async_copy(h.at[sidx.at[j + 1]], mo, go)

                pltpu.async_copy(mc, acc.at[didx.at[j]], sc, add=True)

            @pl.loop(0, nblk)
            def _(b):
                pltpu.sync_copy(src.at[pl.ds(chunk0 + b * BCH, BCH)], sidx)
                pltpu.sync_copy(dst.at[pl.ds(chunk0 + b * BCH, BCH)], didx)
                pltpu.async_copy(h.at[sidx.at[0]], m0, g0)

                @pl.loop(0, BCH, step=2)
                def _(j):
                    phase(j, bufs[0], bufs[1])
                    phase(j + 1, bufs[1], bufs[0])

                pltpu.make_async_copy(m0, acc.at[didx.at[0]], s0).wait()
                pltpu.make_async_copy(m1, acc.at[didx.at[0]], s1).wait()

        @pl.when(c == 0)
        def _():
            edge_pass(h0, s * nch)

        @pl.when(c == 1)
        def _():
            edge_pass(h1, (s if feat_split else NSUB + s) * nch)

        plsc.subcore_barrier()

        @pl.when(c == 0)
        def _():
            pltpu.sync_copy(acc.at[pl.ds(row0, ROWS)],
                            out0.at[pl.ds(row0, ROWS)])

        @pl.when(c == 1)
        def _():
            pltpu.sync_copy(acc.at[pl.ds(row0, ROWS)],
                            out1.at[pl.ds(row0, ROWS)])

    return prop


def _make_degrees():
    mesh = plsc.VectorSubcoreMesh(core_axis_name="c", subcore_axis_name="s")
    out_type = [jax.ShapeDtypeStruct((NP, 16), _f32)] * 4
    nch = EP // (2 * NSUB) // CH
    scratch = [
        pltpu.VMEM_SHARED((NP, 16), _f32),
        pltpu.VMEM_SHARED((NP, 16), _f32),
        pltpu.VMEM((nch, CH), jnp.int32),
        pltpu.VMEM((nch, CH), jnp.int32),
        pltpu.VMEM((CH, 16), _f32),
    ]

    @functools.partial(pl.kernel, out_type=out_type, mesh=mesh,
                       scratch_types=scratch,
                       compiler_params=pltpu.CompilerParams(
                           use_tc_tiling_on_sc=False))
    def degrees(src, dst, zrows, ones_in, o_s0, o_d0, o_s1, o_d1,
                tsrc, tdst, sbuf, dbuf, ones):
        c = lax.axis_index("c")
        s = lax.axis_index("s")
        row0 = s * ROWS
        pltpu.sync_copy(zrows, tsrc.at[pl.ds(row0, ROWS)])
        pltpu.sync_copy(zrows, tdst.at[pl.ds(row0, ROWS)])
        pltpu.sync_copy(ones_in, ones)
        chunk0 = (c * NSUB + s) * nch
        pltpu.sync_copy(src.at[pl.ds(chunk0, nch)], sbuf)
        pltpu.sync_copy(dst.at[pl.ds(chunk0, nch)], dbuf)
        plsc.subcore_barrier()

        @pl.loop(0, nch)
        def _(j):
            pltpu.sync_copy(ones, tsrc.at[sbuf.at[j]], add=True)
            pltpu.sync_copy(ones, tdst.at[dbuf.at[j]], add=True)

        plsc.subcore_barrier()

        @pl.when(c == 0)
        def _():
            pltpu.sync_copy(tsrc.at[pl.ds(row0, ROWS)],
                            o_s0.at[pl.ds(row0, ROWS)])
            pltpu.sync_copy(tdst.at[pl.ds(row0, ROWS)],
                            o_d0.at[pl.ds(row0, ROWS)])

        @pl.when(c == 1)
        def _():
            pltpu.sync_copy(tsrc.at[pl.ds(row0, ROWS)],
                            o_s1.at[pl.ds(row0, ROWS)])
            pltpu.sync_copy(tdst.at[pl.ds(row0, ROWS)],
                            o_d1.at[pl.ds(row0, ROWS)])

    return degrees


_prop128_edge = _make_prop(False, 128)
_prop128_feat = _make_prop(True, 128)
_prop64_edge = _make_prop(False, 64, native_tiling=True)
_degrees = _make_degrees()

_BR = 10240


def _tc_call(body, out_shapes, in_specs, out_specs):
    return pl.pallas_call(
        body,
        grid=(NP // _BR,),
        in_specs=in_specs,
        out_specs=out_specs,
        out_shape=out_shapes,
    )


def _rows(width):
    return pl.BlockSpec((_BR, width), lambda i: (i, 0))


def _full(a, b):
    return pl.BlockSpec((a, b), lambda i: (0, 0))


def _tcA_body(cs0, cs1, cd0, cd1, x, xs_o, dout_o, din_o):
    scnt = cs0[:, 0] + cs1[:, 0]
    dcnt = cd0[:, 0] + cd1[:, 0]
    dout = lax.rsqrt(jnp.maximum(scnt, 1.0))
    din = lax.rsqrt(jnp.maximum(dcnt, 1.0))
    xs_o[...] = x[...] * dout[:, None]
    dout_o[...] = dout[:, None]
    din_o[...] = din[:, None]


def _tcR_body(x, wr, br, res_o):
    res_o[...] = jnp.dot(x[...], wr[...], preferred_element_type=_f32) + br[...]


def _tcB1_body(p0, p1, din, dout, w1, b1, h0_o, h1_o):
    agg = (p0[...] + p1[...]) * din[...]
    t = jnp.maximum(jnp.dot(agg, w1[...], preferred_element_type=_f32)
                    + b1[...], 0.0) * dout[...]
    h0_o[...] = t[:, :128]
    h1_o[...] = t[:, 128:]


def _tcB2_body(a0, a1, din, dout, w2, b2, w3, z3_o):
    agg = jnp.concatenate([a0[...], a1[...]], axis=1) * din[...]
    h2 = jnp.maximum(jnp.dot(agg, w2[...], preferred_element_type=_f32)
                     + b2[...], 0.0)
    z3_o[...] = jnp.dot(h2, w3[...], preferred_element_type=_f32) * dout[...]


def _tcB3_body(p0, p1, din, dout, w4, b3, z4_o):
    h3 = jnp.maximum((p0[...] + p1[...]) * din[...] + b3[...], 0.0)
    z4_o[...] = jnp.dot(h3, w4[...], preferred_element_type=_f32) * dout[...]


def _tcB4_body(p0, p1, din, b4, res, out_o):
    out_o[...] = (p0[...] + p1[...]) * din[...] + b4[...] + res[...]


def kernel(inputs, edge_index, W1, b1, W2, b2, W3, b3, W4, b4, Wr, br):
    x = jnp.pad(inputs, ((0, NP - NN), (0, 0)))
    pad_idx = NN + jnp.arange(EP - NE, dtype=jnp.int32) % (NP - NN)
    src = jnp.concatenate([edge_index[0], pad_idx]).reshape(EP // CH, CH)
    dst = jnp.concatenate([edge_index[1], pad_idx]).reshape(EP // CH, CH)
    z16 = jnp.zeros((ROWS, 16), _f32)
    ones16 = jnp.ones((CH, 16), _f32)

    cs0, cd0, cs1, cd1 = _degrees(src, dst, z16, ones16)

    sdt = jax.ShapeDtypeStruct
    res = _tc_call(
        _tcR_body,
        [sdt((NP, 64), _f32)],
        [_rows(128), _full(128, 64), _full(1, 64)],
        [_rows(64)],
    )(x, Wr, br.reshape(1, 64))[0]

    xs, dout, din = _tc_call(
        _tcA_body,
        [sdt((NP, 128), _f32), sdt((NP, 1), _f32), sdt((NP, 1), _f32)],
        [_rows(16), _rows(16), _rows(16), _rows(16), _rows(128)],
        [_rows(128), _rows(1), _rows(1)],
    )(cs0, cs1, cd0, cd1, x)

    a1p0, a1p1 = _prop128_edge(xs, xs, src, dst)

    h0, h1 = _tc_call(
        _tcB1_body,
        [sdt((NP, 128), _f32), sdt((NP, 128), _f32)],
        [_rows(128), _rows(128), _rows(1), _rows(1),
         _full(128, 256), _full(1, 256)],
        [_rows(128), _rows(128)],
    )(a1p0, a1p1, din, dout, W1, b1.reshape(1, 256))

    a2h0, a2h1 = _prop128_feat(h0, h1, src, dst)

    z3s = _tc_call(
        _tcB2_body,
        [sdt((NP, 128), _f32)],
        [_rows(128), _rows(128), _rows(1), _rows(1),
         _full(256, 256), _full(1, 256), _full(256, 128)],
        [_rows(128)],
    )(a2h0, a2h1, din, dout, W2, b2.reshape(1, 256), W3)[0]

    a3p0, a3p1 = _prop128_edge(z3s, z3s, src, dst)

    z4s = _tc_call(
        _tcB3_body,
        [sdt((NP, 64), _f32)],
        [_rows(128), _rows(128), _rows(1), _rows(1),
         _full(128, 64), _full(1, 128)],
        [_rows(64)],
    )(a3p0, a3p1, din, dout, W4, b3.reshape(1, 128))[0]

    a4p0, a4p1 = _prop64_edge(z4s, z4s, src, dst)

    out = _tc_call(
        _tcB4_body,
        [sdt((NP, 64), _f32)],
        [_rows(64), _rows(64), _rows(1), _full(1, 64), _rows(64)],
        [_rows(64)],
    )(a4p0, a4p1, din, b4.reshape(1, 64), res)[0]

    return out[:NN]


# speedup vs baseline: 1.0131x; 1.0131x over previous
import functools

import jax
import jax.numpy as jnp
from jax import lax
from jax.experimental import pallas as pl
from jax.experimental.pallas import tpu as pltpu
from jax.experimental.pallas import tpu_sc as plsc

NN = 10000
NP = 10240
NE = 320000
EP = 327680
CH = 128
NSUB = 16
ROWS = NP // NSUB

_f32 = jnp.float32


def _make_prop(feat_split: bool, D: int, native_tiling: bool = False):
    mesh = plsc.VectorSubcoreMesh(core_axis_name="c", subcore_axis_name="s")
    out_type = [jax.ShapeDtypeStruct((NP, D), _f32)] * 2
    if feat_split:
        nch = EP // NSUB // CH
    else:
        nch = EP // (2 * NSUB) // CH
    BCH = 40
    nblk = nch // BCH
    scratch = [
        pltpu.VMEM_SHARED((NP, D), _f32),
        pltpu.VMEM((BCH, CH), jnp.int32),
        pltpu.VMEM((BCH, CH), jnp.int32),
        pltpu.VMEM((CH, D), _f32),
        pltpu.VMEM((CH, D), _f32),
        pltpu.SemaphoreType.DMA,
        pltpu.SemaphoreType.DMA,
        pltpu.SemaphoreType.DMA,
        pltpu.SemaphoreType.DMA,
    ]

    cp = (pltpu.CompilerParams(use_tc_tiling_on_sc=False)
          if native_tiling else None)

    @functools.partial(pl.kernel, out_type=out_type, mesh=mesh,
                       scratch_types=scratch, compiler_params=cp)
    def prop(h0, h1, src, dst, out0, out1,
             acc, sidx, didx, m0, m1, g0, g1, s0, s1):
        c = lax.axis_index("c")
        s = lax.axis_index("s")
        row0 = s * ROWS
        zv = jnp.zeros((16,), _f32)

        @pl.loop(0, CH)
        def _(r):
            @pl.loop(0, D, step=16)
            def _(k):
                m0[r, pl.ds(k, 16)] = zv

        @pl.loop(0, ROWS, step=CH)
        def _(r):
            pltpu.sync_copy(m0, acc.at[pl.ds(row0 + r, CH)])

        plsc.subcore_barrier()

        def edge_pass(h, chunk0):
            bufs = ((m0, g0, s0), (m1, g1, s1))

            def phase(j, cur, oth):
                mc, gc, sc = cur
                mo, go, so = oth
                pltpu.make_async_copy(h.at[sidx.at[j]], mc, gc).wait()

                @pl.when(j + 1 < BCH)
                def _():
                    @pl.when(j >= 1)
                    def _():
                        pltpu.make_async_copy(mo, acc.at[didx.at[0]],
                                              so).wait()
                    pltpu.async_copy(h.at[sidx.at[j + 1]], mo, go)

                pltpu.async_copy(mc, acc.at[didx.at[j]], sc, add=True)

            @pl.loop(0, nblk)
            def _(b):
                pltpu.sync_copy(src.at[pl.ds(chunk0 + b * BCH, BCH)], sidx)
                pltpu.sync_copy(dst.at[pl.ds(chunk0 + b * BCH, BCH)], didx)
                pltpu.async_copy(h.at[sidx.at[0]], m0, g0)

                @pl.loop(0, BCH, step=2)
                def _(j):
                    phase(j, bufs[0], bufs[1])
                    phase(j + 1, bufs[1], bufs[0])

                pltpu.make_async_copy(m0, acc.at[didx.at[0]], s0).wait()
                pltpu.make_async_copy(m1, acc.at[didx.at[0]], s1).wait()

        @pl.when(c == 0)
        def _():
            edge_pass(h0, s * nch)

        @pl.when(c == 1)
        def _():
            edge_pass(h1, (s if feat_split else NSUB + s) * nch)

        plsc.subcore_barrier()

        @pl.when(c == 0)
        def _():
            pltpu.sync_copy(acc.at[pl.ds(row0, ROWS)],
                            out0.at[pl.ds(row0, ROWS)])

        @pl.when(c == 1)
        def _():
            pltpu.sync_copy(acc.at[pl.ds(row0, ROWS)],
                            out1.at[pl.ds(row0, ROWS)])

    return prop


def _make_degrees():
    mesh = plsc.VectorSubcoreMesh(core_axis_name="c", subcore_axis_name="s")
    out_type = [jax.ShapeDtypeStruct((NP, 16), _f32)] * 4
    nch = EP // (2 * NSUB) // CH
    scratch = [
        pltpu.VMEM_SHARED((NP, 16), _f32),
        pltpu.VMEM_SHARED((NP, 16), _f32),
        pltpu.VMEM((nch, CH), jnp.int32),
        pltpu.VMEM((nch, CH), jnp.int32),
        pltpu.VMEM((CH, 16), _f32),
    ]

    @functools.partial(pl.kernel, out_type=out_type, mesh=mesh,
                       scratch_types=scratch,
                       compiler_params=pltpu.CompilerParams(
                           use_tc_tiling_on_sc=False))
    def degrees(src, dst, zrows, ones_in, o_s0, o_d0, o_s1, o_d1,
                tsrc, tdst, sbuf, dbuf, ones):
        c = lax.axis_index("c")
        s = lax.axis_index("s")
        row0 = s * ROWS
        pltpu.sync_copy(zrows, tsrc.at[pl.ds(row0, ROWS)])
        pltpu.sync_copy(zrows, tdst.at[pl.ds(row0, ROWS)])
        pltpu.sync_copy(ones_in, ones)
        chunk0 = (c * NSUB + s) * nch
        pltpu.sync_copy(src.at[pl.ds(chunk0, nch)], sbuf)
        pltpu.sync_copy(dst.at[pl.ds(chunk0, nch)], dbuf)
        plsc.subcore_barrier()

        @pl.loop(0, nch)
        def _(j):
            pltpu.sync_copy(ones, tsrc.at[sbuf.at[j]], add=True)
            pltpu.sync_copy(ones, tdst.at[dbuf.at[j]], add=True)

        plsc.subcore_barrier()

        @pl.when(c == 0)
        def _():
            pltpu.sync_copy(tsrc.at[pl.ds(row0, ROWS)],
                            o_s0.at[pl.ds(row0, ROWS)])
            pltpu.sync_copy(tdst.at[pl.ds(row0, ROWS)],
                            o_d0.at[pl.ds(row0, ROWS)])

        @pl.when(c == 1)
        def _():
            pltpu.sync_copy(tsrc.at[pl.ds(row0, ROWS)],
                            o_s1.at[pl.ds(row0, ROWS)])
            pltpu.sync_copy(tdst.at[pl.ds(row0, ROWS)],
                            o_d1.at[pl.ds(row0, ROWS)])

    return degrees


_prop128_edge = _make_prop(False, 128)
_prop128_feat = _make_prop(True, 128)
_prop64_edge = _make_prop(False, 64, native_tiling=True)
_degrees = _make_degrees()

_BR = 5120


def _tc_call(body, out_shapes, in_specs, out_specs):
    return pl.pallas_call(
        body,
        grid=(NP // _BR,),
        in_specs=in_specs,
        out_specs=out_specs,
        out_shape=out_shapes,
    )


def _rows(width):
    return pl.BlockSpec((_BR, width), lambda i: (i, 0))


def _full(a, b):
    return pl.BlockSpec((a, b), lambda i: (0, 0))


def _tcA_body(cs0, cs1, cd0, cd1, x, xs_o, dout_o, din_o):
    scnt = cs0[:, 0] + cs1[:, 0]
    dcnt = cd0[:, 0] + cd1[:, 0]
    dout = lax.rsqrt(jnp.maximum(scnt, 1.0))
    din = lax.rsqrt(jnp.maximum(dcnt, 1.0))
    xs_o[...] = x[...] * dout[:, None]
    dout_o[...] = dout[:, None]
    din_o[...] = din[:, None]


def _tcR_body(x, wr, br, res_o):
    res_o[...] = jnp.dot(x[...], wr[...], preferred_element_type=_f32) + br[...]


def _tcB1_body(p0, p1, din, dout, w1, b1, h0_o, h1_o):
    agg = (p0[...] + p1[...]) * din[...]
    t = jnp.maximum(jnp.dot(agg, w1[...], preferred_element_type=_f32)
                    + b1[...], 0.0) * dout[...]
    h0_o[...] = t[:, :128]
    h1_o[...] = t[:, 128:]


def _tcB2_body(a0, a1, din, dout, w2, b2, w3, z3_o):
    agg = jnp.concatenate([a0[...], a1[...]], axis=1) * din[...]
    h2 = jnp.maximum(jnp.dot(agg, w2[...], preferred_element_type=_f32)
                     + b2[...], 0.0)
    z3_o[...] = jnp.dot(h2, w3[...], preferred_element_type=_f32) * dout[...]


def _tcB3_body(p0, p1, din, dout, w4, b3, z4_o):
    h3 = jnp.maximum((p0[...] + p1[...]) * din[...] + b3[...], 0.0)
    z4_o[...] = jnp.dot(h3, w4[...], preferred_element_type=_f32) * dout[...]


def _tcB4_body(p0, p1, din, b4, res, out_o):
    out_o[...] = (p0[...] + p1[...]) * din[...] + b4[...] + res[...]


def kernel(inputs, edge_index, W1, b1, W2, b2, W3, b3, W4, b4, Wr, br):
    x = jnp.pad(inputs, ((0, NP - NN), (0, 0)))
    pad_idx = NN + jnp.arange(EP - NE, dtype=jnp.int32) % (NP - NN)
    src = jnp.concatenate([edge_index[0], pad_idx]).reshape(EP // CH, CH)
    dst = jnp.concatenate([edge_index[1], pad_idx]).reshape(EP // CH, CH)
    z16 = jnp.zeros((ROWS, 16), _f32)
    ones16 = jnp.ones((CH, 16), _f32)

    cs0, cd0, cs1, cd1 = _degrees(src, dst, z16, ones16)

    sdt = jax.ShapeDtypeStruct
    res = _tc_call(
        _tcR_body,
        [sdt((NP, 64), _f32)],
        [_rows(128), _full(128, 64), _full(1, 64)],
        [_rows(64)],
    )(x, Wr, br.reshape(1, 64))[0]

    xs, dout, din = _tc_call(
        _tcA_body,
        [sdt((NP, 128), _f32), sdt((NP, 1), _f32), sdt((NP, 1), _f32)],
        [_rows(16), _rows(16), _rows(16), _rows(16), _rows(128)],
        [_rows(128), _rows(1), _rows(1)],
    )(cs0, cs1, cd0, cd1, x)

    a1p0, a1p1 = _prop128_edge(xs, xs, src, dst)

    h0, h1 = _tc_call(
        _tcB1_body,
        [sdt((NP, 128), _f32), sdt((NP, 128), _f32)],
        [_rows(128), _rows(128), _rows(1), _rows(1),
         _full(128, 256), _full(1, 256)],
        [_rows(128), _rows(128)],
    )(a1p0, a1p1, din, dout, W1, b1.reshape(1, 256))

    a2h0, a2h1 = _prop128_feat(h0, h1, src, dst)

    z3s = _tc_call(
        _tcB2_body,
        [sdt((NP, 128), _f32)],
        [_rows(128), _rows(128), _rows(1), _rows(1),
         _full(256, 256), _full(1, 256), _full(256, 128)],
        [_rows(128)],
    )(a2h0, a2h1, din, dout, W2, b2.reshape(1, 256), W3)[0]

    a3p0, a3p1 = _prop128_edge(z3s, z3s, src, dst)

    z4s = _tc_call(
        _tcB3_body,
        [sdt((NP, 64), _f32)],
        [_rows(128), _rows(128), _rows(1), _rows(1),
         _full(128, 64), _full(1, 128)],
        [_rows(64)],
    )(a3p0, a3p1, din, dout, W4, b3.reshape(1, 128))[0]

    a4p0, a4p1 = _prop64_edge(z4s, z4s, src, dst)

    out = _tc_call(
        _tcB4_body,
        [sdt((NP, 64), _f32)],
        [_rows(64), _rows(64), _rows(1), _full(1, 64), _rows(64)],
        [_rows(64)],
    )(a4p0, a4p1, din, b4.reshape(1, 64), res)[0]

    return out[:NN]
